# baseline (speedup 1.0000x reference)
import functools

import jax
import jax.numpy as jnp
from jax import lax
from jax.experimental import pallas as pl
from jax.experimental.pallas import tpu as pltpu
from jax.experimental.pallas import tpu_sc as plsc

VOCAB = 100000
EMBED = 128
HIDDEN = 256
CLASSES = 64
BATCH = 4096
SEQ = 50

NC = 2
NS = 16
NW = NC * NS
B_PER_W = BATCH // NW
G = 2
NG = B_PER_W // G
LANES = 16
NCHUNK = EMBED // LANES


def _sc_pool(x_grouped, emb):
    mesh = plsc.VectorSubcoreMesh(core_axis_name="c", subcore_axis_name="s")

    ng = x_grouped.shape[1]
    b_per_w = ng * G
    bsz = NW * b_per_w
    NBUF = 4

    @functools.partial(
        pl.kernel,
        mesh=mesh,
        out_type=jax.ShapeDtypeStruct((bsz, EMBED), jnp.float32),
        scratch_types=[
            pltpu.VMEM((ng, G * SEQ), jnp.int32),
            pltpu.VMEM((b_per_w, EMBED), jnp.float32),
        ]
        + [pltpu.VMEM((G * SEQ, EMBED), jnp.float32) for _ in range(NBUF)]
        + [pltpu.SemaphoreType.DMA for _ in range(NBUF)],
    )
    def k(x_hbm, emb_hbm, out_hbm, idx_v, pooled_v, *bufs_and_sems):
        bufs = bufs_and_sems[:NBUF]
        sems = bufs_and_sems[NBUF:]
        wid = lax.axis_index("s") * NC + lax.axis_index("c")
        pltpu.sync_copy(x_hbm.at[wid], idx_v)

        def start(g, b):
            pltpu.async_copy(emb_hbm.at[idx_v.at[g]], bufs[b], sems[b])

        def wait(g, b):
            pltpu.make_async_copy(emb_hbm.at[idx_v.at[g]], bufs[b], sems[b]).wait()

        def accumulate(g, b):
            rows_v = bufs[b]
            for r in range(G):
                def tok_body(j, acc):
                    return tuple(
                        acc[d] + rows_v[r * SEQ + j, pl.ds(d * LANES, LANES)]
                        for d in range(NCHUNK)
                    )
                acc0 = tuple(
                    rows_v[r * SEQ, pl.ds(d * LANES, LANES)]
                    for d in range(NCHUNK)
                )
                acc = lax.fori_loop(1, SEQ, tok_body, acc0)
                for d in range(NCHUNK):
                    pooled_v[g * G + r, pl.ds(d * LANES, LANES)] = acc[d]

        for b in range(NBUF):
            start(b, b)

        def outer(i, _):
            g0 = i * NBUF
            for b in range(NBUF):
                g = g0 + b
                wait(g, b)
                accumulate(g, b)
                nxt = g + NBUF

                @pl.when(nxt < ng)
                def _():
                    start(nxt, b)
            return ()

        lax.fori_loop(0, ng // NBUF, outer, ())
        pltpu.sync_copy(pooled_v, out_hbm.at[pl.ds(wid * b_per_w, b_per_w)])

    return k(x_grouped, emb)


def _tc_mlp(pooled, W1, b1, W2, b2):
    BB = 1024

    def body(p_ref, w1_ref, b1_ref, w2_ref, b2_ref, o_ref):
        m = p_ref[...] * (1.0 / SEQ)
        h = lax.dot_general(m, w1_ref[...], (((1,), (1,)), ((), ())),
                            preferred_element_type=jnp.float32) + b1_ref[...]
        z = lax.dot_general(h, w2_ref[...], (((1,), (1,)), ((), ())),
                            preferred_element_type=jnp.float32) + b2_ref[...]
        z = z - jnp.max(z, axis=-1, keepdims=True)
        e = jnp.exp(z)
        o_ref[...] = e / jnp.sum(e, axis=-1, keepdims=True)

    bsz = pooled.shape[0]
    return pl.pallas_call(
        body,
        grid=(bsz // BB,),
        in_specs=[
            pl.BlockSpec((BB, EMBED), lambda i: (i, 0)),
            pl.BlockSpec((HIDDEN, EMBED), lambda i: (0, 0)),
            pl.BlockSpec((1, HIDDEN), lambda i: (0, 0)),
            pl.BlockSpec((CLASSES, HIDDEN), lambda i: (0, 0)),
            pl.BlockSpec((1, CLASSES), lambda i: (0, 0)),
        ],
        out_specs=pl.BlockSpec((BB, CLASSES), lambda i: (i, 0)),
        out_shape=jax.ShapeDtypeStruct((bsz, CLASSES), jnp.float32),
    )(pooled, W1, b1.reshape(1, HIDDEN), W2, b2.reshape(1, CLASSES))


def kernel(x, emb, W1, b1, W2, b2):
    x_grouped = x.reshape(NW, NG, G * SEQ)
    pooled = _sc_pool(x_grouped, emb)
    return _tc_mlp(pooled, W1, b1, W2, b2)


# speedup vs baseline: 1.0201x; 1.0201x over previous
import functools

import jax
import jax.numpy as jnp
from jax import lax
from jax.experimental import pallas as pl
from jax.experimental.pallas import tpu as pltpu
from jax.experimental.pallas import tpu_sc as plsc

VOCAB = 100000
EMBED = 128
HIDDEN = 256
CLASSES = 64
BATCH = 4096
SEQ = 50

NC = 2
NS = 16
NW = NC * NS
B_PER_W = BATCH // NW
G = 2
NG = B_PER_W // G
LANES = 16
NCHUNK = EMBED // LANES


def _sc_pool(x_grouped, emb):
    mesh = plsc.VectorSubcoreMesh(core_axis_name="c", subcore_axis_name="s")

    ng = x_grouped.shape[1]
    b_per_w = ng * G
    bsz = NW * b_per_w
    NBUF = 4

    @functools.partial(
        pl.kernel,
        mesh=mesh,
        out_type=jax.ShapeDtypeStruct((bsz, EMBED), jnp.float32),
        scratch_types=[
            pltpu.VMEM((ng, G * SEQ), jnp.int32),
            pltpu.VMEM((b_per_w, EMBED), jnp.float32),
        ]
        + [pltpu.VMEM((G * SEQ, EMBED), jnp.float32) for _ in range(NBUF)]
        + [pltpu.SemaphoreType.DMA for _ in range(NBUF)],
    )
    def k(x_hbm, emb_hbm, out_hbm, idx_v, pooled_v, *bufs_and_sems):
        bufs = bufs_and_sems[:NBUF]
        sems = bufs_and_sems[NBUF:]
        wid = lax.axis_index("s") * NC + lax.axis_index("c")
        pltpu.sync_copy(x_hbm.at[wid], idx_v)

        def start(g, b):
            pltpu.async_copy(emb_hbm.at[idx_v.at[g]], bufs[b], sems[b])

        def wait(g, b):
            pltpu.make_async_copy(emb_hbm.at[idx_v.at[g]], bufs[b], sems[b]).wait()

        def accumulate(g, b):
            rows_v = bufs[b]
            for r in range(G):
                def tok_body(j, acc):
                    return tuple(
                        acc[d] + rows_v[r * SEQ + j, pl.ds(d * LANES, LANES)]
                        for d in range(NCHUNK)
                    )
                acc0 = tuple(
                    rows_v[r * SEQ, pl.ds(d * LANES, LANES)]
                    for d in range(NCHUNK)
                )
                acc = lax.fori_loop(1, SEQ, tok_body, acc0)
                for d in range(NCHUNK):
                    pooled_v[g * G + r, pl.ds(d * LANES, LANES)] = acc[d]

        for b in range(NBUF):
            start(b, b)

        def outer(i, _):
            g0 = i * NBUF
            for b in range(NBUF):
                g = g0 + b
                wait(g, b)
                accumulate(g, b)
                nxt = g + NBUF

                @pl.when(nxt < ng)
                def _():
                    start(nxt, b)
            return ()

        lax.fori_loop(0, ng // NBUF, outer, ())
        pltpu.sync_copy(pooled_v, out_hbm.at[pl.ds(wid * b_per_w, b_per_w)])

    return k(x_grouped, emb)


def _tc_mlp(pooled, W1, b1, W2, b2):
    BB = 2048

    def body(p_ref, w1_ref, b1_ref, w2_ref, b2_ref, o_ref):
        m = p_ref[...] * (1.0 / SEQ)
        h = lax.dot_general(m, w1_ref[...], (((1,), (1,)), ((), ())),
                            preferred_element_type=jnp.float32) + b1_ref[...]
        z = lax.dot_general(h, w2_ref[...], (((1,), (1,)), ((), ())),
                            preferred_element_type=jnp.float32) + b2_ref[...]
        z = z - jnp.max(z, axis=-1, keepdims=True)
        e = jnp.exp(z)
        o_ref[...] = e / jnp.sum(e, axis=-1, keepdims=True)

    bsz = pooled.shape[0]
    return pl.pallas_call(
        body,
        grid=(bsz // BB,),
        in_specs=[
            pl.BlockSpec((BB, EMBED), lambda i: (i, 0)),
            pl.BlockSpec((HIDDEN, EMBED), lambda i: (0, 0)),
            pl.BlockSpec((1, HIDDEN), lambda i: (0, 0)),
            pl.BlockSpec((CLASSES, HIDDEN), lambda i: (0, 0)),
            pl.BlockSpec((1, CLASSES), lambda i: (0, 0)),
        ],
        out_specs=pl.BlockSpec((BB, CLASSES), lambda i: (i, 0)),
        out_shape=jax.ShapeDtypeStruct((bsz, CLASSES), jnp.float32),
    )(pooled, W1, b1.reshape(1, HIDDEN), W2, b2.reshape(1, CLASSES))


def kernel(x, emb, W1, b1, W2, b2):
    x_grouped = x.reshape(NW, NG, G * SEQ)
    pooled = _sc_pool(x_grouped, emb)
    return _tc_mlp(pooled, W1, b1, W2, b2)
